# baseline (speedup 1.0000x reference)
import functools

import jax
import jax.numpy as jnp
from jax import lax
from jax.experimental import pallas as pl
from jax.experimental.pallas import tpu as pltpu
from jax.experimental.pallas import tpu_sc as plsc

N = 10000
C = 128
E = 320000

NC = 2
NS = 16
NW = NC * NS

PAD_N = 10112
E_PAD = 327680
CHUNK = 128
EPT = E_PAD // NW
NCHUNK = EPT // CHUNK
ACHUNK = 64
AROWS = 81
ACT = 2 * AROWS
E_AGG = NW * ACT * ACHUNK
RPT = PAD_N // NS
DEG_PAD = 10240
DPT = DEG_PAD // NS
NBUF = 3
DEG_Q = 8

_MESH = plsc.VectorSubcoreMesh(core_axis_name="c", subcore_axis_name="s")


@functools.partial(
    pl.kernel,
    mesh=_MESH,
    out_type=jax.ShapeDtypeStruct((NC, DEG_PAD), jnp.float32),
    scratch_types=[
        pltpu.VMEM((NCHUNK, CHUNK), jnp.int32),
        pltpu.VMEM((CHUNK,), jnp.float32),
        pltpu.VMEM((DPT,), jnp.float32),
        pltpu.VMEM_SHARED((DEG_PAD,), jnp.float32),
        pltpu.SemaphoreType.DMA,
    ],
)
def _deg_kernel(dst_hbm, out_hbm, dst_v, ones_v, zer_v, deg_sh, sem):
    cid = lax.axis_index("c")
    sid = lax.axis_index("s")
    wid = cid * NS + sid
    for j in range(DPT // 16):
        zer_v[pl.ds(j * 16, 16)] = jnp.zeros((16,), jnp.float32)
    for j in range(CHUNK // 16):
        ones_v[pl.ds(j * 16, 16)] = jnp.ones((16,), jnp.float32)
    pltpu.sync_copy(zer_v, deg_sh.at[pl.ds(sid * DPT, DPT)])
    pltpu.sync_copy(dst_hbm.at[wid], dst_v)
    plsc.subcore_barrier()

    def step(j, carry):
        for b in range(DEG_Q):
            pltpu.async_copy(ones_v, deg_sh.at[dst_v.at[j * DEG_Q + b]], sem,
                             add=True)
        for b in range(DEG_Q):
            pltpu.make_async_copy(ones_v, deg_sh.at[dst_v.at[j * DEG_Q + b]],
                                  sem).wait()
        return carry

    lax.fori_loop(0, NCHUNK // DEG_Q, step, 0)
    plsc.subcore_barrier()
    pltpu.sync_copy(deg_sh.at[pl.ds(sid * DPT, DPT)],
                    out_hbm.at[cid, pl.ds(sid * DPT, DPT)])


def _transform_body(x_ref, w_ref, dp_ref, g_ref):
    h = jnp.dot(x_ref[...], w_ref[...], preferred_element_type=jnp.float32)
    deg = dp_ref[0, :N] + dp_ref[1, :N] + 1.0
    disq = lax.rsqrt(deg)
    g_ref[:N, :] = h * disq[:, None]
    g_ref[N:, :] = jnp.zeros((PAD_N - N, C), jnp.float32)


_transform = pl.pallas_call(
    _transform_body,
    out_shape=jax.ShapeDtypeStruct((PAD_N, C), jnp.float32),
)


@functools.partial(
    pl.kernel,
    mesh=_MESH,
    out_type=jax.ShapeDtypeStruct((NC, PAD_N, C), jnp.float32),
    scratch_types=[
        pltpu.VMEM((AROWS, 2 * ACHUNK), jnp.int32),
        pltpu.VMEM((AROWS, 2 * ACHUNK), jnp.int32),
        pltpu.VMEM((NBUF, ACHUNK, C), jnp.float32),
        pltpu.VMEM_SHARED((PAD_N, C), jnp.float32),
    ] + [pltpu.SemaphoreType.DMA] * (2 * NBUF),
)
def _agg_kernel(src_hbm, dst_hbm, g_hbm, out_hbm,
                src_v, dst_v, rows_v, acc_sh, *sems):
    sem_g = sems[:NBUF]
    sem_s = sems[NBUF:]
    cid = lax.axis_index("c")
    sid = lax.axis_index("s")
    wid = cid * NS + sid
    pltpu.sync_copy(g_hbm.at[pl.ds(sid * RPT, RPT)],
                    acc_sh.at[pl.ds(sid * RPT, RPT)])
    pltpu.sync_copy(src_hbm.at[wid], src_v)
    pltpu.sync_copy(dst_hbm.at[wid], dst_v)
    plsc.subcore_barrier()

    def s_idx(r, h):
        return src_v.at[r, pl.ds(h * ACHUNK, ACHUNK)]

    def d_idx(r, h):
        return dst_v.at[r, pl.ds(h * ACHUNK, ACHUNK)]

    def gather(r, h, s):
        pltpu.async_copy(g_hbm.at[s_idx(r, h)], rows_v.at[s], sem_g[s])

    def gather_wait(r, h, s):
        pltpu.make_async_copy(g_hbm.at[s_idx(r, h)], rows_v.at[s],
                              sem_g[s]).wait()

    def scat(r, h, s):
        pltpu.async_copy(rows_v.at[s], acc_sh.at[d_idx(r, h)], sem_s[s],
                         add=True)

    def scat_wait(r, h, s):
        pltpu.make_async_copy(rows_v.at[s], acc_sh.at[d_idx(r, h)],
                              sem_s[s]).wait()

    gather(0, 0, 0)
    gather(0, 1, 1)

    def step(j, carry):
        for o in range(6):
            r = 3 * j + o // 2
            h = o % 2
            s = o % 3
            gather_wait(r, h, s)
            scat(r, h, s)
            pr = 3 * j + (o - 1) // 2 if o >= 1 else 3 * j - 3 + 2
            ph = (o - 1) % 2
            ps = (o + 2) % 3
            if o == 0:
                @pl.when(j >= 1)
                def _():
                    scat_wait(pr, ph, ps)
            else:
                scat_wait(pr, ph, ps)
            nr = 3 * j + (o + 2) // 2
            if o < 4:
                gather(nr, h, ps)
            else:
                @pl.when(j + 1 < AROWS // 3)
                def _():
                    gather(nr, h, ps)
        return carry

    lax.fori_loop(0, AROWS // 3, step, 0)
    scat_wait(AROWS - 1, 1, (ACT - 1) % 3)
    plsc.subcore_barrier()
    pltpu.sync_copy(acc_sh.at[pl.ds(sid * RPT, RPT)],
                    out_hbm.at[cid, pl.ds(sid * RPT, RPT)])


def _finalize_body(s_ref, g_ref, dp_ref, b_ref, o_ref):
    deg = dp_ref[0, :N] + dp_ref[1, :N] + 1.0
    disq = lax.rsqrt(deg)
    tot = s_ref[0, :N, :] + s_ref[1, :N, :] - g_ref[:N, :]
    o_ref[...] = tot * disq[:, None] + b_ref[...][None, :]


_finalize = pl.pallas_call(
    _finalize_body,
    out_shape=jax.ShapeDtypeStruct((N, C), jnp.float32),
)


def kernel(x, edge_index, W, b):
    src = edge_index[0].astype(jnp.int32)
    dst = edge_index[1].astype(jnp.int32)
    pad_d = jnp.full((E_PAD - E,), N, jnp.int32)
    pad_a = jnp.full((E_AGG - E,), N, jnp.int32)
    dst_deg = jnp.concatenate([dst, pad_d]).reshape(NW, NCHUNK, CHUNK)
    src_agg = jnp.concatenate([src, pad_a]).reshape(NW, AROWS, 2 * ACHUNK)
    dst_agg = jnp.concatenate([dst, pad_a]).reshape(NW, AROWS, 2 * ACHUNK)

    deg_parts = _deg_kernel(dst_deg)
    g_pad = _transform(x, W, deg_parts)
    s_parts = _agg_kernel(src_agg, dst_agg, g_pad)
    return _finalize(s_parts, g_pad, deg_parts, b)


# speedup vs baseline: 1.5903x; 1.5903x over previous
import functools

import jax
import jax.numpy as jnp
from jax import lax
from jax.experimental import pallas as pl
from jax.experimental.pallas import tpu as pltpu
from jax.experimental.pallas import tpu_sc as plsc

N = 10000
C = 128
E = 320000

NC = 2
NS = 16
NW = NC * NS

PAD_N = 10112
E_PAD = 327680
CHUNK = 128
EPT = E_PAD // NW
NCHUNK = EPT // CHUNK
ACHUNK = 64
AROWS = EPT // (2 * ACHUNK)
E_AGG = E_PAD
RPT = PAD_N // NS
DEG_PAD = 10240
DPT = DEG_PAD // NS
NBUF = 2
DEG_Q = 8

_MESH = plsc.VectorSubcoreMesh(core_axis_name="c", subcore_axis_name="s")


@functools.partial(
    pl.kernel,
    mesh=_MESH,
    out_type=jax.ShapeDtypeStruct((NC, DEG_PAD), jnp.float32),
    scratch_types=[
        pltpu.VMEM((NCHUNK, CHUNK), jnp.int32),
        pltpu.VMEM((CHUNK,), jnp.float32),
        pltpu.VMEM((DPT,), jnp.float32),
        pltpu.VMEM_SHARED((DEG_PAD,), jnp.float32),
        pltpu.SemaphoreType.DMA,
    ],
)
def _deg_kernel(dst_hbm, out_hbm, dst_v, ones_v, zer_v, deg_sh, sem):
    cid = lax.axis_index("c")
    sid = lax.axis_index("s")
    wid = cid * NS + sid
    for j in range(DPT // 16):
        zer_v[pl.ds(j * 16, 16)] = jnp.zeros((16,), jnp.float32)
    for j in range(CHUNK // 16):
        ones_v[pl.ds(j * 16, 16)] = jnp.ones((16,), jnp.float32)
    pltpu.sync_copy(zer_v, deg_sh.at[pl.ds(sid * DPT, DPT)])
    pltpu.sync_copy(dst_hbm.at[wid], dst_v)
    plsc.subcore_barrier()

    def step(j, carry):
        for b in range(DEG_Q):
            pltpu.async_copy(ones_v, deg_sh.at[dst_v.at[j * DEG_Q + b]], sem,
                             add=True)
        for b in range(DEG_Q):
            pltpu.make_async_copy(ones_v, deg_sh.at[dst_v.at[j * DEG_Q + b]],
                                  sem).wait()
        return carry

    lax.fori_loop(0, NCHUNK // DEG_Q, step, 0)
    plsc.subcore_barrier()
    pltpu.sync_copy(deg_sh.at[pl.ds(sid * DPT, DPT)],
                    out_hbm.at[cid, pl.ds(sid * DPT, DPT)])


def _transform_body(x_ref, w_ref, dp_ref, g_ref):
    h = jnp.dot(x_ref[...], w_ref[...], preferred_element_type=jnp.float32)
    deg = dp_ref[0, :N] + dp_ref[1, :N] + 1.0
    disq = lax.rsqrt(deg)
    g_ref[:N, :] = h * disq[:, None]
    g_ref[N:, :] = jnp.zeros((PAD_N - N, C), jnp.float32)


_transform = pl.pallas_call(
    _transform_body,
    out_shape=jax.ShapeDtypeStruct((PAD_N, C), jnp.float32),
)


@functools.partial(
    pl.kernel,
    mesh=_MESH,
    out_type=jax.ShapeDtypeStruct((NC, PAD_N, C), jnp.float32),
    scratch_types=[
        pltpu.VMEM((AROWS, 2 * ACHUNK), jnp.int32),
        pltpu.VMEM((AROWS, 2 * ACHUNK), jnp.int32),
        pltpu.VMEM((NBUF, ACHUNK, C), jnp.float32),
        pltpu.VMEM_SHARED((PAD_N, C), jnp.float32),
    ] + [pltpu.SemaphoreType.DMA] * (2 * NBUF),
)
def _agg_kernel(src_hbm, dst_hbm, g_hbm, out_hbm,
                src_v, dst_v, rows_v, acc_sh, *sems):
    sem_g = sems[:NBUF]
    sem_s = sems[NBUF:]
    cid = lax.axis_index("c")
    sid = lax.axis_index("s")
    wid = cid * NS + sid
    pltpu.sync_copy(g_hbm.at[pl.ds(sid * RPT, RPT)],
                    acc_sh.at[pl.ds(sid * RPT, RPT)])
    pltpu.sync_copy(src_hbm.at[wid], src_v)
    pltpu.sync_copy(dst_hbm.at[wid], dst_v)
    plsc.subcore_barrier()

    def s_idx(j, h):
        return src_v.at[j, pl.ds(h * ACHUNK, ACHUNK)]

    def d_idx(j, h):
        return dst_v.at[j, pl.ds(h * ACHUNK, ACHUNK)]

    def gather(j, h):
        pltpu.async_copy(g_hbm.at[s_idx(j, h)], rows_v.at[h], sem_g[h])

    def gather_wait(j, h):
        pltpu.make_async_copy(g_hbm.at[s_idx(j, h)], rows_v.at[h],
                              sem_g[h]).wait()

    def scat(j, h):
        pltpu.async_copy(rows_v.at[h], acc_sh.at[d_idx(j, h)], sem_s[h],
                         add=True)

    def scat_wait(j, h):
        pltpu.make_async_copy(rows_v.at[h], acc_sh.at[d_idx(j, h)],
                              sem_s[h]).wait()

    gather(0, 0)

    def step(j, carry):
        gather_wait(j, 0)
        scat(j, 0)

        @pl.when(j >= 1)
        def _():
            scat_wait(j - 1, 1)

        gather(j, 1)
        gather_wait(j, 1)
        scat(j, 1)
        scat_wait(j, 0)

        @pl.when(j + 1 < AROWS)
        def _():
            gather(j + 1, 0)

        return carry

    lax.fori_loop(0, AROWS, step, 0)
    scat_wait(AROWS - 1, 1)
    plsc.subcore_barrier()
    pltpu.sync_copy(acc_sh.at[pl.ds(sid * RPT, RPT)],
                    out_hbm.at[cid, pl.ds(sid * RPT, RPT)])


def _finalize_body(s_ref, g_ref, dp_ref, b_ref, o_ref):
    deg = dp_ref[0, :N] + dp_ref[1, :N] + 1.0
    disq = lax.rsqrt(deg)
    tot = s_ref[0, :N, :] + s_ref[1, :N, :] - g_ref[:N, :]
    o_ref[...] = tot * disq[:, None] + b_ref[...][None, :]


_finalize = pl.pallas_call(
    _finalize_body,
    out_shape=jax.ShapeDtypeStruct((N, C), jnp.float32),
)


def kernel(x, edge_index, W, b):
    src = edge_index[0].astype(jnp.int32)
    dst = edge_index[1].astype(jnp.int32)
    pad = jnp.full((E_PAD - E,), N, jnp.int32)
    src_p = jnp.concatenate([src, pad])
    dst_p = jnp.concatenate([dst, pad])
    dst_deg = dst_p.reshape(NW, NCHUNK, CHUNK)
    src_agg = src_p.reshape(NW, AROWS, 2 * ACHUNK)
    dst_agg = dst_p.reshape(NW, AROWS, 2 * ACHUNK)

    deg_parts = _deg_kernel(dst_deg)
    g_pad = _transform(x, W, deg_parts)
    s_parts = _agg_kernel(src_agg, dst_agg, g_pad)
    return _finalize(s_parts, g_pad, deg_parts, b)


# speedup vs baseline: 1.5922x; 1.0012x over previous
import functools

import jax
import jax.numpy as jnp
from jax import lax
from jax.experimental import pallas as pl
from jax.experimental.pallas import tpu as pltpu
from jax.experimental.pallas import tpu_sc as plsc

N = 10000
C = 128
E = 320000

NC = 2
NS = 16
NW = NC * NS

PAD_N = 10112
E_PAD = 327680
CHUNK = 128
EPT = E_PAD // NW
NCHUNK = EPT // CHUNK
ACHUNK = 64
AROWS = EPT // (2 * ACHUNK)
E_AGG = E_PAD
RPT = PAD_N // NS
DEG_PAD = 10240
DPT = DEG_PAD // NS
NBUF = 2
DEG_Q = 8

_MESH = plsc.VectorSubcoreMesh(core_axis_name="c", subcore_axis_name="s")


@functools.partial(
    pl.kernel,
    mesh=_MESH,
    out_type=jax.ShapeDtypeStruct((NC, DEG_PAD), jnp.float32),
    scratch_types=[
        pltpu.VMEM((NCHUNK, CHUNK), jnp.int32),
        pltpu.VMEM((CHUNK,), jnp.float32),
        pltpu.VMEM((DPT,), jnp.float32),
        pltpu.VMEM_SHARED((DEG_PAD,), jnp.float32),
        pltpu.SemaphoreType.DMA,
    ],
)
def _deg_kernel(dst_hbm, out_hbm, dst_v, ones_v, zer_v, deg_sh, sem):
    cid = lax.axis_index("c")
    sid = lax.axis_index("s")
    wid = cid * NS + sid
    for j in range(DPT // 16):
        zer_v[pl.ds(j * 16, 16)] = jnp.zeros((16,), jnp.float32)
    for j in range(CHUNK // 16):
        ones_v[pl.ds(j * 16, 16)] = jnp.ones((16,), jnp.float32)
    pltpu.sync_copy(zer_v, deg_sh.at[pl.ds(sid * DPT, DPT)])
    pltpu.sync_copy(dst_hbm.at[wid], dst_v)
    plsc.subcore_barrier()

    def step(j, carry):
        for b in range(DEG_Q):
            pltpu.async_copy(ones_v, deg_sh.at[dst_v.at[j * DEG_Q + b]], sem,
                             add=True)
        for b in range(DEG_Q):
            pltpu.make_async_copy(ones_v, deg_sh.at[dst_v.at[j * DEG_Q + b]],
                                  sem).wait()
        return carry

    lax.fori_loop(0, NCHUNK // DEG_Q, step, 0)
    plsc.subcore_barrier()
    pltpu.sync_copy(deg_sh.at[pl.ds(sid * DPT, DPT)],
                    out_hbm.at[cid, pl.ds(sid * DPT, DPT)])


def _transform_body(x_ref, w_ref, dp_ref, g_ref, gb_ref):
    h = jnp.dot(x_ref[...], w_ref[...], preferred_element_type=jnp.float32)
    deg = dp_ref[0, :N] + dp_ref[1, :N] + 1.0
    disq = lax.rsqrt(deg)
    g = h * disq[:, None]
    g_ref[...] = g
    gb_ref[:N, :] = g
    gb_ref[N:, :] = jnp.zeros((PAD_N - N, C), jnp.float32)


_transform = pl.pallas_call(
    _transform_body,
    out_shape=(
        jax.ShapeDtypeStruct((N, C), jnp.float32),
        jax.ShapeDtypeStruct((PAD_N, C), jnp.float32),
    ),
)


@functools.partial(
    pl.kernel,
    mesh=_MESH,
    out_type=jax.ShapeDtypeStruct((NC, PAD_N, C), jnp.float32),
    scratch_types=[
        pltpu.VMEM((AROWS, 2 * ACHUNK), jnp.int32),
        pltpu.VMEM((AROWS, 2 * ACHUNK), jnp.int32),
        pltpu.VMEM((NBUF, ACHUNK, C), jnp.float32),
        pltpu.VMEM_SHARED((PAD_N, C), jnp.float32),
    ] + [pltpu.SemaphoreType.DMA] * (2 * NBUF),
)
def _agg_kernel(src_hbm, dst_hbm, g_hbm, z_hbm, out_hbm,
                src_v, dst_v, rows_v, acc_sh, *sems):
    sem_g = sems[:NBUF]
    sem_s = sems[NBUF:]
    cid = lax.axis_index("c")
    sid = lax.axis_index("s")
    wid = cid * NS + sid
    pltpu.sync_copy(z_hbm.at[pl.ds(sid * RPT, RPT)],
                    acc_sh.at[pl.ds(sid * RPT, RPT)])
    pltpu.sync_copy(src_hbm.at[wid], src_v)
    pltpu.sync_copy(dst_hbm.at[wid], dst_v)
    plsc.subcore_barrier()

    def s_idx(j, h):
        return src_v.at[j, pl.ds(h * ACHUNK, ACHUNK)]

    def d_idx(j, h):
        return dst_v.at[j, pl.ds(h * ACHUNK, ACHUNK)]

    def gather(j, h):
        pltpu.async_copy(g_hbm.at[s_idx(j, h)], rows_v.at[h], sem_g[h])

    def gather_wait(j, h):
        pltpu.make_async_copy(g_hbm.at[s_idx(j, h)], rows_v.at[h],
                              sem_g[h]).wait()

    def scat(j, h):
        pltpu.async_copy(rows_v.at[h], acc_sh.at[d_idx(j, h)], sem_s[h],
                         add=True)

    def scat_wait(j, h):
        pltpu.make_async_copy(rows_v.at[h], acc_sh.at[d_idx(j, h)],
                              sem_s[h]).wait()

    gather(0, 0)

    def step(j, carry):
        gather_wait(j, 0)
        scat(j, 0)

        @pl.when(j >= 1)
        def _():
            scat_wait(j - 1, 1)

        gather(j, 1)
        gather_wait(j, 1)
        scat(j, 1)
        scat_wait(j, 0)

        @pl.when(j + 1 < AROWS)
        def _():
            gather(j + 1, 0)

        return carry

    lax.fori_loop(0, AROWS, step, 0)
    scat_wait(AROWS - 1, 1)
    plsc.subcore_barrier()
    pltpu.sync_copy(acc_sh.at[pl.ds(sid * RPT, RPT)],
                    out_hbm.at[cid, pl.ds(sid * RPT, RPT)])


def _finalize_body(s_ref, g_ref, dp_ref, b_ref, o_ref):
    deg = dp_ref[0, :N] + dp_ref[1, :N] + 1.0
    disq = lax.rsqrt(deg)
    tot = (s_ref[0, :N, :].astype(jnp.float32)
           + s_ref[1, :N, :].astype(jnp.float32) + g_ref[...])
    o_ref[...] = tot * disq[:, None] + b_ref[...][None, :]


_finalize = pl.pallas_call(
    _finalize_body,
    out_shape=jax.ShapeDtypeStruct((N, C), jnp.float32),
)


def kernel(x, edge_index, W, b):
    src = edge_index[0].astype(jnp.int32)
    dst = edge_index[1].astype(jnp.int32)
    pad = jnp.full((E_PAD - E,), N, jnp.int32)
    src_p = jnp.concatenate([src, pad])
    dst_p = jnp.concatenate([dst, pad])
    dst_deg = dst_p.reshape(NW, NCHUNK, CHUNK)
    src_agg = src_p.reshape(NW, AROWS, 2 * ACHUNK)
    dst_agg = dst_p.reshape(NW, AROWS, 2 * ACHUNK)

    deg_parts = _deg_kernel(dst_deg)
    g_f32, g_pad = _transform(x, W, deg_parts)
    zeros_f32 = jnp.zeros((PAD_N, C), jnp.float32)
    s_parts = _agg_kernel(src_agg, dst_agg, g_pad, zeros_f32)
    return _finalize(s_parts, g_f32, deg_parts, b)
